# baseline (speedup 1.0000x reference)
import functools

import jax
import jax.numpy as jnp
from jax.experimental import pallas as pl
from jax.experimental.pallas import tpu as pltpu
from jax.experimental.shard_map import shard_map
from jax.sharding import PartitionSpec as PSpec

B, P, D = 4096, 8192, 64
BLOCK_B = 512

_NDEV = len(jax.devices())
_B_LOCAL = B // _NDEV


def _centroid_kernel(x_ref, c_ref, mask_ref, out_ref, cn_ref, cm_ref, bias_ref):
    @pl.when(pl.program_id(0) == 0)
    def _prep():
        c = c_ref[...]
        m = mask_ref[...]
        cn = c / (jnp.sqrt(jnp.sum(c * c, axis=-1, keepdims=True)) + 1e-12)
        cn_ref[...] = cn.astype(jnp.bfloat16)
        cm_ref[...] = (c * m.reshape(P, 1)).astype(jnp.bfloat16)
        bias_ref[...] = jnp.where(m > 0, 0.0, -1e9).astype(jnp.float32)

    x = x_ref[...]
    xn = x / (jnp.sqrt(jnp.sum(x * x, axis=-1, keepdims=True)) + 1e-12)

    sim = jax.lax.dot_general(
        xn.astype(jnp.bfloat16), cn_ref[...], (((1,), (1,)), ((), ())),
        preferred_element_type=jnp.float32)
    e = jnp.exp(sim + bias_ref[...])
    s = jnp.sum(e, axis=-1, keepdims=True)

    ctx = jax.lax.dot_general(
        e.astype(jnp.bfloat16), cm_ref[...], (((1,), (0,)), ((), ())),
        preferred_element_type=jnp.float32)
    out_ref[...] = ctx / s


def _centroid_local(x, centroid_emb, maskf):
    return pl.pallas_call(
        _centroid_kernel,
        grid=(_B_LOCAL // BLOCK_B,),
        in_specs=[
            pl.BlockSpec((BLOCK_B, D), lambda i: (i, 0)),
            pl.BlockSpec((P, D), lambda i: (0, 0)),
            pl.BlockSpec((1, P), lambda i: (0, 0)),
        ],
        out_specs=pl.BlockSpec((BLOCK_B, D), lambda i: (i, 0)),
        out_shape=jax.ShapeDtypeStruct((x.shape[0], D), jnp.float32),
        scratch_shapes=[
            pltpu.VMEM((P, D), jnp.bfloat16),
            pltpu.VMEM((P, D), jnp.bfloat16),
            pltpu.VMEM((1, P), jnp.float32),
        ],
    )(x, centroid_emb, maskf)


@jax.jit
def kernel(x, centroid_emb, active_mask):
    maskf = active_mask.astype(jnp.float32).reshape(1, P)
    if _NDEV == 1:
        return _centroid_local(x, centroid_emb, maskf)
    mesh = jax.make_mesh((_NDEV,), ("b",))
    x = jax.reshard(x, jax.NamedSharding(mesh, PSpec("b", None)))
    centroid_emb = jax.reshard(
        centroid_emb, jax.NamedSharding(mesh, PSpec(None, None)))
    maskf = jax.reshard(maskf, jax.NamedSharding(mesh, PSpec(None, None)))
    fn = shard_map(
        _centroid_local,
        mesh=mesh,
        in_specs=(PSpec("b", None), PSpec(None, None), PSpec(None, None)),
        out_specs=PSpec("b", None),
        check_rep=False,
    )
    return fn(x, centroid_emb, maskf)


# speedup vs baseline: 8.4636x; 8.4636x over previous
import jax
import jax.numpy as jnp
from jax.experimental import pallas as pl
from jax.experimental.pallas import tpu as pltpu

B, P, D = 4096, 8192, 64
BLOCK_B = 512


def _centroid_kernel(x_ref, c_ref, mask_ref, out_ref, cn_ref, cm_ref, bias_ref):
    @pl.when(pl.program_id(0) == 0)
    def _prep():
        c = c_ref[...]
        m = mask_ref[...]
        cn = c / (jnp.sqrt(jnp.sum(c * c, axis=-1, keepdims=True)) + 1e-12)
        cn_ref[...] = cn.astype(jnp.bfloat16)
        cm_ref[...] = (c * m.reshape(P, 1)).astype(jnp.bfloat16)
        bias_ref[...] = jnp.where(m > 0, 0.0, -1e9).astype(jnp.float32)

    x = x_ref[...]
    xn = x / (jnp.sqrt(jnp.sum(x * x, axis=-1, keepdims=True)) + 1e-12)

    sim = jax.lax.dot_general(
        xn.astype(jnp.bfloat16), cn_ref[...], (((1,), (1,)), ((), ())),
        preferred_element_type=jnp.float32)
    e = jnp.exp(sim + bias_ref[...])
    s = jnp.sum(e, axis=-1, keepdims=True)

    ctx = jax.lax.dot_general(
        e, cm_ref[...], (((1,), (0,)), ((), ())),
        preferred_element_type=jnp.float32)
    out_ref[...] = ctx / s


@jax.jit
def kernel(x, centroid_emb, active_mask):
    maskf = active_mask.astype(jnp.float32).reshape(1, P)
    return pl.pallas_call(
        _centroid_kernel,
        grid=(B // BLOCK_B,),
        in_specs=[
            pl.BlockSpec((BLOCK_B, D), lambda i: (i, 0)),
            pl.BlockSpec((P, D), lambda i: (0, 0)),
            pl.BlockSpec((1, P), lambda i: (0, 0)),
        ],
        out_specs=pl.BlockSpec((BLOCK_B, D), lambda i: (i, 0)),
        out_shape=jax.ShapeDtypeStruct((B, D), jnp.float32),
        scratch_shapes=[
            pltpu.VMEM((P, D), jnp.bfloat16),
            pltpu.VMEM((P, D), jnp.bfloat16),
            pltpu.VMEM((1, P), jnp.float32),
        ],
    )(x, centroid_emb, maskf)
